# baseline (speedup 1.0000x reference)
import functools

import jax
import jax.numpy as jnp
from jax import lax
from jax.experimental import pallas as pl
from jax.experimental.pallas import tpu as pltpu
from jax.experimental.pallas import tpu_sc as plsc

N = 8192
K = 8192
D = 64
TN = 512
TK = K
NI = N // TN

_PREC = lax.Precision.DEFAULT


def _dist_body(emb_ref, w_ref, b_ref, z_ref, d_ref, idx_ref, cb_ref,
               c2_ref, ids_ref):
    i = pl.program_id(0)

    @pl.when(i == 0)
    def _():
        cb0 = lax.dot_general(
            emb_ref[...], w_ref[...], (((1,), (1,)), ((), ())),
            precision=_PREC, preferred_element_type=jnp.float32) + b_ref[...]
        cb_ref[...] = cb0
        c2_ref[...] = jnp.sum(cb0 * cb0, axis=1).reshape(1, K)
        ids_ref[...] = lax.broadcasted_iota(jnp.int32, (1, K), 1).astype(jnp.float32)

    z = jnp.transpose(z_ref[...])
    c = cb_ref[...]
    z2 = jnp.sum(z * z, axis=1, keepdims=True)
    c2 = c2_ref[...]
    dot = lax.dot_general(z, c, (((1,), (1,)), ((), ())),
                          precision=_PREC, preferred_element_type=jnp.float32)
    dist = z2 + c2 - 2.0 * dot
    d_ref[...] = dist
    row_min = jnp.min(dist, axis=1)
    masked = jnp.where(dist == row_min[:, None], ids_ref[...], jnp.float32(2 * K))
    idx_ref[...] = jnp.min(masked, axis=1).astype(jnp.int32).reshape(1, 1, TN)


_dist_call = pl.pallas_call(
    _dist_body,
    grid=(NI,),
    in_specs=[
        pl.BlockSpec((K, D), lambda i: (0, 0)),
        pl.BlockSpec((D, D), lambda i: (0, 0)),
        pl.BlockSpec((1, D), lambda i: (0, 0)),
        pl.BlockSpec((D, TN), lambda i: (i // 2, i % 2)),
    ],
    out_specs=[
        pl.BlockSpec((TN, K), lambda i: (i, 0)),
        pl.BlockSpec((1, 1, TN), lambda i: (i, 0, 0)),
        pl.BlockSpec((K, D), lambda i: (0, 0)),
    ],
    out_shape=[
        jax.ShapeDtypeStruct((N, K), jnp.float32),
        jax.ShapeDtypeStruct((NI, 1, TN), jnp.int32),
        jax.ShapeDtypeStruct((K, D), jnp.float32),
    ],
    scratch_shapes=[
        pltpu.VMEM((1, K), jnp.float32),
        pltpu.VMEM((1, K), jnp.float32),
    ],
)


_SC_NC = 2
_SC_NS = 16
_NW = _SC_NC * _SC_NS
_BPW = N // _NW


def _gather_body(table_hbm, idx_hbm, zeros_hbm, out_hbm, cnt_hbm,
                 idx_v, rows_v, ones_v, shared_cnt, sem):
    cid = lax.axis_index("c")
    sid = lax.axis_index("s")
    wid = sid * _SC_NC + cid
    base = wid * _BPW
    pltpu.sync_copy(idx_hbm.at[pl.ds(base, _BPW)], idx_v)
    gather = pltpu.async_copy(table_hbm.at[idx_v], rows_v, sem)

    def ofill(i, _):
        ones_v[pl.ds(i * 16, 16)] = jnp.ones((16,), jnp.float32)
        return 0

    lax.fori_loop(0, _BPW // 16, ofill, 0)

    @pl.when(sid == 0)
    def _():
        pltpu.sync_copy(zeros_hbm, shared_cnt)

    plsc.subcore_barrier()
    pltpu.sync_copy(ones_v, shared_cnt.at[idx_v], add=True)
    gather.wait()
    pltpu.sync_copy(rows_v, out_hbm.at[pl.ds(base, _BPW)])
    plsc.subcore_barrier()

    @pl.when(sid == 0)
    def _():
        pltpu.sync_copy(shared_cnt, cnt_hbm.at[cid])


@functools.cache
def _get_gather_call():
    return pl.kernel(
        _gather_body,
        mesh=plsc.VectorSubcoreMesh(core_axis_name="c", subcore_axis_name="s"),
        out_type=[
            jax.ShapeDtypeStruct((N, D), jnp.float32),
            jax.ShapeDtypeStruct((_SC_NC, K), jnp.float32),
        ],
        scratch_types=[
            pltpu.VMEM((_BPW,), jnp.int32),
            pltpu.VMEM((_BPW, D), jnp.float32),
            pltpu.VMEM((_BPW,), jnp.float32),
            pltpu.VMEM_SHARED((K,), jnp.float32),
            pltpu.SemaphoreType.DMA,
        ],
        compiler_params=pltpu.CompilerParams(use_tc_tiling_on_sc=False),
    )


_B = 8
_HW = 1024


def _stats_body(z_ref, q_ref, cnt_ref, out_ref, loss_ref, ppl_ref):
    acc = jnp.zeros((), jnp.float32)
    for bi in range(_B):
        qbt = jnp.transpose(q_ref[pl.ds(bi * _HW, _HW), :])
        zb = z_ref[pl.ds(bi * D, D), :]
        diff = qbt - zb
        acc = acc + jnp.sum(diff * diff)
        out_ref[pl.ds(bi * D, D), :] = qbt
    m = acc * (1.0 / (N * D))
    loss_ref[...] = jnp.reshape(1.0 * m + 0.25 * m, (1, 1))
    cnt = cnt_ref[...]
    e = (cnt[0:1, :] + cnt[1:2, :]) * (1.0 / N)
    ent = jnp.sum(e * jnp.log(e + 1e-10))
    ppl_ref[...] = jnp.reshape(jnp.exp(-ent), (1, 1))


_stats_call = pl.pallas_call(
    _stats_body,
    out_shape=[
        jax.ShapeDtypeStruct((_B * D, _HW), jnp.float32),
        jax.ShapeDtypeStruct((1, 1), jnp.float32),
        jax.ShapeDtypeStruct((1, 1), jnp.float32),
    ],
)


def kernel(z, emb_weight, W, b):
    z2d = z.reshape(_B * D, _HW)
    d, idx3, codebook = _dist_call(emb_weight, W, b.reshape(1, D), z2d)
    idx = idx3.reshape(N)
    z_q, counts = _get_gather_call()(codebook, idx, jnp.zeros((K,), jnp.float32))
    out2d, loss11, ppl11 = _stats_call(z2d, z_q, counts)
    z_q_out = out2d.reshape(z.shape)
    loss = loss11[0, 0]
    perplexity = ppl11[0, 0]
    perplexity_loss = jnp.zeros((), jnp.float32)
    return (z_q_out, loss, d, perplexity, idx, perplexity_loss)


# speedup vs baseline: 1.1330x; 1.1330x over previous
import functools

import jax
import jax.numpy as jnp
from jax import lax
from jax.experimental import pallas as pl
from jax.experimental.pallas import tpu as pltpu
from jax.experimental.pallas import tpu_sc as plsc

N = 8192
K = 8192
D = 64
TN = 512
TK = K
NI = N // TN

_PREC = lax.Precision.DEFAULT


def _dist_body(emb_ref, w_ref, b_ref, z_ref, d_ref, idx_ref, cb_ref,
               c2_ref, ids_ref):
    i = pl.program_id(0)

    @pl.when(i == 0)
    def _():
        cb0 = lax.dot_general(
            emb_ref[...], w_ref[...], (((1,), (1,)), ((), ())),
            precision=_PREC, preferred_element_type=jnp.float32) + b_ref[...]
        cb_ref[...] = cb0
        c2_ref[...] = jnp.sum(cb0 * cb0, axis=1).reshape(1, K)
        ids_ref[...] = lax.broadcasted_iota(jnp.int32, (1, K), 1).astype(jnp.float32)

    z = z_ref[...]
    c = cb_ref[...]
    z2 = jnp.sum(z * z, axis=1, keepdims=True)
    c2 = c2_ref[...]
    dot = lax.dot_general(z, c, (((1,), (1,)), ((), ())),
                          precision=_PREC, preferred_element_type=jnp.float32)
    dist = z2 + c2 - 2.0 * dot
    d_ref[...] = dist
    row_min = jnp.min(dist, axis=1)
    masked = jnp.where(dist == row_min[:, None], ids_ref[...], jnp.float32(2 * K))
    idx_ref[...] = jnp.min(masked, axis=1).astype(jnp.int32).reshape(1, 1, TN)


_dist_call = pl.pallas_call(
    _dist_body,
    grid=(NI,),
    in_specs=[
        pl.BlockSpec((K, D), lambda i: (0, 0)),
        pl.BlockSpec((D, D), lambda i: (0, 0)),
        pl.BlockSpec((1, D), lambda i: (0, 0)),
        pl.BlockSpec((TN, D), lambda i: (i, 0)),
    ],
    out_specs=[
        pl.BlockSpec((TN, K), lambda i: (i, 0)),
        pl.BlockSpec((1, 1, TN), lambda i: (i, 0, 0)),
        pl.BlockSpec((K, D), lambda i: (0, 0)),
    ],
    out_shape=[
        jax.ShapeDtypeStruct((N, K), jnp.float32),
        jax.ShapeDtypeStruct((NI, 1, TN), jnp.int32),
        jax.ShapeDtypeStruct((K, D), jnp.float32),
    ],
    scratch_shapes=[
        pltpu.VMEM((1, K), jnp.float32),
        pltpu.VMEM((1, K), jnp.float32),
    ],
)


_SC_NC = 2
_SC_NS = 16
_NW = _SC_NC * _SC_NS
_BPW = N // _NW


def _gather_body(table_hbm, idx_hbm, zeros_hbm, out_hbm, cnt_hbm,
                 idx_v, rows_v, ones_v, shared_cnt, sem):
    cid = lax.axis_index("c")
    sid = lax.axis_index("s")
    wid = sid * _SC_NC + cid
    base = wid * _BPW
    pltpu.sync_copy(idx_hbm.at[pl.ds(base, _BPW)], idx_v)
    gather = pltpu.async_copy(table_hbm.at[idx_v], rows_v, sem)

    def ofill(i, _):
        ones_v[pl.ds(i * 16, 16)] = jnp.ones((16,), jnp.float32)
        return 0

    lax.fori_loop(0, _BPW // 16, ofill, 0)

    @pl.when(sid == 0)
    def _():
        pltpu.sync_copy(zeros_hbm, shared_cnt)

    plsc.subcore_barrier()
    pltpu.sync_copy(ones_v, shared_cnt.at[idx_v], add=True)
    gather.wait()
    pltpu.sync_copy(rows_v, out_hbm.at[pl.ds(base, _BPW)])
    plsc.subcore_barrier()

    @pl.when(sid == 0)
    def _():
        pltpu.sync_copy(shared_cnt, cnt_hbm.at[cid])


@functools.cache
def _get_gather_call():
    return pl.kernel(
        _gather_body,
        mesh=plsc.VectorSubcoreMesh(core_axis_name="c", subcore_axis_name="s"),
        out_type=[
            jax.ShapeDtypeStruct((N, D), jnp.float32),
            jax.ShapeDtypeStruct((_SC_NC, K), jnp.float32),
        ],
        scratch_types=[
            pltpu.VMEM((_BPW,), jnp.int32),
            pltpu.VMEM((_BPW, D), jnp.float32),
            pltpu.VMEM((_BPW,), jnp.float32),
            pltpu.VMEM_SHARED((K,), jnp.float32),
            pltpu.SemaphoreType.DMA,
        ],
        compiler_params=pltpu.CompilerParams(use_tc_tiling_on_sc=False),
    )


_B = 8
_HW = 1024


def _stats_body(z_ref, q_ref, cnt_ref, loss_ref, ppl_ref):
    diff = q_ref[...] - z_ref[...]
    m = jnp.mean(diff * diff)
    loss_ref[...] = jnp.reshape(1.0 * m + 0.25 * m, (1, 1))
    cnt = cnt_ref[...]
    e = (cnt[0:1, :] + cnt[1:2, :]) * (1.0 / N)
    ent = jnp.sum(e * jnp.log(e + 1e-10))
    ppl_ref[...] = jnp.reshape(jnp.exp(-ent), (1, 1))


_stats_call = pl.pallas_call(
    _stats_body,
    out_shape=[
        jax.ShapeDtypeStruct((1, 1), jnp.float32),
        jax.ShapeDtypeStruct((1, 1), jnp.float32),
    ],
)


def kernel(z, emb_weight, W, b):
    zt = jnp.transpose(z, (0, 2, 3, 1))
    z_flat = zt.reshape(-1, D)
    d, idx3, codebook = _dist_call(emb_weight, W, b.reshape(1, D), z_flat)
    idx = idx3.reshape(N)
    z_q, counts = _get_gather_call()(codebook, idx, jnp.zeros((K,), jnp.float32))
    loss11, ppl11 = _stats_call(z_flat, z_q, counts)
    z_q_out = jnp.transpose(z_q.reshape(zt.shape), (0, 3, 1, 2))
    loss = loss11[0, 0]
    perplexity = ppl11[0, 0]
    perplexity_loss = jnp.zeros((), jnp.float32)
    return (z_q_out, loss, d, perplexity, idx, perplexity_loss)


# speedup vs baseline: 1.1347x; 1.0015x over previous
import functools

import jax
import jax.numpy as jnp
from jax import lax
from jax.experimental import pallas as pl
from jax.experimental.pallas import tpu as pltpu
from jax.experimental.pallas import tpu_sc as plsc

N = 8192
K = 8192
D = 64
TN = 512
TK = K
NI = N // TN

_PREC = lax.Precision.DEFAULT


def _dist_body(emb_ref, w_ref, b_ref, z_ref, d_ref, idx_ref, cb_ref,
               c2_ref, ids_ref):
    i = pl.program_id(0)

    @pl.when(i == 0)
    def _():
        cb0 = lax.dot_general(
            emb_ref[...], w_ref[...], (((1,), (1,)), ((), ())),
            precision=_PREC, preferred_element_type=jnp.float32) + b_ref[...]
        cb_ref[...] = cb0
        c2_ref[...] = jnp.sum(cb0 * cb0, axis=1).reshape(1, K)
        ids_ref[...] = lax.broadcasted_iota(jnp.int32, (1, K), 1).astype(jnp.float32)

    z = z_ref[...]
    c = cb_ref[...]
    z2 = jnp.sum(z * z, axis=1, keepdims=True)
    c2 = c2_ref[...]
    dot = lax.dot_general(z, c, (((1,), (1,)), ((), ())),
                          precision=_PREC, preferred_element_type=jnp.float32)
    dist = z2 + c2 - 2.0 * dot
    d_ref[...] = dist
    row_min = jnp.min(dist, axis=1)
    masked = jnp.where(dist == row_min[:, None], ids_ref[...], jnp.float32(2 * K))
    idx_ref[...] = jnp.min(masked, axis=1).astype(jnp.int32).reshape(1, 1, TN)


_dist_call = pl.pallas_call(
    _dist_body,
    grid=(NI,),
    in_specs=[
        pl.BlockSpec((K, D), lambda i: (0, 0)),
        pl.BlockSpec((D, D), lambda i: (0, 0)),
        pl.BlockSpec((1, D), lambda i: (0, 0)),
        pl.BlockSpec((TN, D), lambda i: (i, 0)),
    ],
    out_specs=[
        pl.BlockSpec((TN, K), lambda i: (i, 0)),
        pl.BlockSpec((1, 1, TN), lambda i: (i, 0, 0)),
        pl.BlockSpec((K, D), lambda i: (0, 0)),
    ],
    out_shape=[
        jax.ShapeDtypeStruct((N, K), jnp.float32),
        jax.ShapeDtypeStruct((NI, 1, TN), jnp.int32),
        jax.ShapeDtypeStruct((K, D), jnp.float32),
    ],
    scratch_shapes=[
        pltpu.VMEM((1, K), jnp.float32),
        pltpu.VMEM((1, K), jnp.float32),
    ],
)


_SC_NC = 2
_SC_NS = 16
_NW = _SC_NC * _SC_NS
_BPW = N // _NW


def _gather_body(table_hbm, idx_hbm, zeros_hbm, out_hbm, cnt_hbm,
                 idx_v, rows_v, ones_v, shared_cnt, sem):
    cid = lax.axis_index("c")
    sid = lax.axis_index("s")
    wid = sid * _SC_NC + cid
    base = wid * _BPW
    pltpu.sync_copy(idx_hbm.at[pl.ds(base, _BPW)], idx_v)
    gather = pltpu.async_copy(table_hbm.at[idx_v], rows_v, sem)

    def ofill(i, _):
        ones_v[pl.ds(i * 16, 16)] = jnp.ones((16,), jnp.float32)
        return 0

    lax.fori_loop(0, _BPW // 16, ofill, 0)

    @pl.when(sid == 0)
    def _():
        pltpu.sync_copy(zeros_hbm, shared_cnt)

    plsc.subcore_barrier()
    pltpu.sync_copy(ones_v, shared_cnt.at[idx_v], add=True)
    gather.wait()
    pltpu.sync_copy(rows_v, out_hbm.at[pl.ds(base, _BPW)])
    plsc.subcore_barrier()

    @pl.when(sid == 0)
    def _():
        pltpu.sync_copy(shared_cnt, cnt_hbm.at[cid])


@functools.cache
def _get_gather_call():
    return pl.kernel(
        _gather_body,
        mesh=plsc.VectorSubcoreMesh(core_axis_name="c", subcore_axis_name="s"),
        out_type=[
            jax.ShapeDtypeStruct((N, D), jnp.float32),
            jax.ShapeDtypeStruct((_SC_NC, K), jnp.float32),
        ],
        scratch_types=[
            pltpu.VMEM((_BPW,), jnp.int32),
            pltpu.VMEM((_BPW, D), jnp.float32),
            pltpu.VMEM((_BPW,), jnp.float32),
            pltpu.VMEM_SHARED((K,), jnp.float32),
            pltpu.SemaphoreType.DMA,
        ],
        compiler_params=pltpu.CompilerParams(use_tc_tiling_on_sc=False),
    )


def _stats_body(z_ref, q_ref, cnt_ref, loss_ref, ppl_ref):
    diff = q_ref[...] - z_ref[...]
    m = jnp.mean(diff * diff)
    loss_ref[...] = jnp.reshape(1.0 * m + 0.25 * m, (1, 1))
    cnt = cnt_ref[...]
    e = (cnt[0:1, :] + cnt[1:2, :]) * (1.0 / N)
    ent = jnp.sum(e * jnp.log(e + 1e-10))
    ppl_ref[...] = jnp.reshape(jnp.exp(-ent), (1, 1))


_stats_call = pl.pallas_call(
    _stats_body,
    out_shape=[
        jax.ShapeDtypeStruct((1, 1), jnp.float32),
        jax.ShapeDtypeStruct((1, 1), jnp.float32),
    ],
)


def kernel(z, emb_weight, W, b):
    zt = jnp.transpose(z, (0, 2, 3, 1))
    z_flat = zt.reshape(-1, D)
    d, idx3, codebook = _dist_call(emb_weight, W, b.reshape(1, D), z_flat)
    idx = idx3.reshape(N)
    z_q, counts = _get_gather_call()(codebook, idx, jnp.zeros((K,), jnp.float32))
    loss11, ppl11 = _stats_call(z_flat, z_q, counts)
    z_q_out = jnp.transpose(z_q.reshape(zt.shape), (0, 3, 1, 2))
    loss = loss11[0, 0]
    perplexity = ppl11[0, 0]
    perplexity_loss = jnp.zeros((), jnp.float32)
    return (z_q_out, loss, d, perplexity, idx, perplexity_loss)
